# Initial kernel scaffold; baseline (speedup 1.0000x reference)
import jax
import jax.numpy as jnp
from jax import lax
from jax.experimental import pallas as pl
from jax.experimental.pallas import tpu as pltpu
from jax.experimental.pallas import tpu_sc as plsc

N = 10000
E = 160000
D = 256
H = 4
ED = 16
FD = 512
Q = 3
K = 512
G = 64
HD = H * D

NC = 2
NS = 16
NW = NC * NS
EPW = E // NW
CHUNK = 40
NCHUNK = EPW // CHUNK


def _mm2_body(x_ref, wl_ref, bl_ref, wr_ref, br_ref, o1_ref, o2_ref):
    xb = x_ref[...]
    o1_ref[...] = jnp.dot(xb, wl_ref[...], preferred_element_type=jnp.float32) + bl_ref[...]
    o2_ref[...] = jnp.dot(xb, wr_ref[...], preferred_element_type=jnp.float32) + br_ref[...]


def _mm2(x, W_l, b_l, W_r, b_r):
    BN = 2000
    return pl.pallas_call(
        _mm2_body,
        grid=(N // BN,),
        in_specs=[
            pl.BlockSpec((BN, D), lambda i: (i, 0)),
            pl.BlockSpec((D, HD), lambda i: (0, 0)),
            pl.BlockSpec((1, HD), lambda i: (0, 0)),
            pl.BlockSpec((D, HD), lambda i: (0, 0)),
            pl.BlockSpec((1, HD), lambda i: (0, 0)),
        ],
        out_specs=[
            pl.BlockSpec((BN, HD), lambda i: (i, 0)),
            pl.BlockSpec((BN, HD), lambda i: (i, 0)),
        ],
        out_shape=[
            jax.ShapeDtypeStruct((N, HD), jnp.float32),
            jax.ShapeDtypeStruct((N, HD), jnp.float32),
        ],
    )(x, W_l, b_l.reshape(1, HD), W_r, b_r.reshape(1, HD))


def _mm1_body(a_ref, w_ref, o_ref):
    o_ref[...] = jnp.dot(a_ref[...], w_ref[...], preferred_element_type=jnp.float32)


def _ea_mm(edge_attr, W_e):
    BE = 4000
    return pl.pallas_call(
        _mm1_body,
        grid=(E // BE,),
        in_specs=[
            pl.BlockSpec((BE, ED), lambda i: (i, 0)),
            pl.BlockSpec((ED, HD), lambda i: (0, 0)),
        ],
        out_specs=pl.BlockSpec((BE, HD), lambda i: (i, 0)),
        out_shape=jax.ShapeDtypeStruct((E, HD), jnp.float32),
    )(edge_attr, W_e)


def _sc_pass1_body(ei, xl, xr, ea, att, s_out, denom_out,
                   srcb, dstb, xlb, xrb, eab, attv, srow, zrow, denom_sh,
                   sem1, sem2, sem3, sem4):
    cid = lax.axis_index("c")
    sid = lax.axis_index("s")
    wid = sid * NC + cid
    base_e = wid * EPW

    pltpu.sync_copy(att.at[pl.ds(0, HD)], attv)

    def _zr(i, _):
        zrow[i, :] = jnp.zeros((16,), jnp.float32)
        return 0
    lax.fori_loop(0, 1000, _zr, 0)

    @pl.when(sid == 0)
    def _():
        def _zd(i, _):
            pltpu.sync_copy(zrow, denom_sh.at[pl.ds(i * 1000, 1000)])
            return 0
        lax.fori_loop(0, N // 1000, _zd, 0)

    plsc.subcore_barrier()

    lane = lax.broadcasted_iota(jnp.int32, (16,), 0)
    hmask = lane < H

    def _chunk(k, _):
        gbase = base_e + k * CHUNK
        pltpu.sync_copy(ei.at[0, pl.ds(gbase, CHUNK)], srcb)
        pltpu.sync_copy(ei.at[1, pl.ds(gbase, CHUNK)], dstb)
        c1 = pltpu.async_copy(xl.at[srcb], xlb, sem1)
        c2 = pltpu.async_copy(xr.at[dstb], xrb, sem2)
        c3 = pltpu.async_copy(ea.at[pl.ds(gbase, CHUNK)], eab, sem3)
        c1.wait()
        c2.wait()
        c3.wait()

        def _edge(i, _):
            for h in range(H):
                def _dd(j, acc):
                    off = h * D + j * 16
                    z = xlb[i, pl.ds(off, 16)] + xrb[i, pl.ds(off, 16)] + eab[i, pl.ds(off, 16)]
                    z = jnp.maximum(z, 0.2 * z)
                    return acc + z * attv[pl.ds(off, 16)]
                acc = lax.fori_loop(0, D // 16, _dd, jnp.zeros((16,), jnp.float32))
                srow[i, h] = jnp.sum(acc)
            return 0
        lax.fori_loop(0, CHUNK, _edge, 0)

        def _exp(i, _):
            v = srow[i, :]
            v = jnp.where(hmask, v, 0.0)
            srow[i, :] = jnp.where(hmask, jnp.exp(v), 0.0)
            return 0
        lax.fori_loop(0, CHUNK, _exp, 0)

        pltpu.sync_copy(srow, s_out.at[pl.ds(gbase, CHUNK)])
        pltpu.async_copy(srow, denom_sh.at[dstb], sem4, add=True).wait()
        return 0

    lax.fori_loop(0, NCHUNK, _chunk, 0)

    plsc.subcore_barrier()

    @pl.when(sid == 0)
    def _():
        def _dump(i, _):
            pltpu.sync_copy(denom_sh.at[pl.ds(i * 1000, 1000)],
                            denom_out.at[cid, pl.ds(i * 1000, 1000)])
            return 0
        lax.fori_loop(0, N // 1000, _dump, 0)


def _sc_pass1(edge_index, xl, xr, ea, att_flat):
    mesh = plsc.VectorSubcoreMesh(core_axis_name="c", subcore_axis_name="s",
                                  num_cores=NC, num_subcores=NS)
    f = pl.kernel(
        _sc_pass1_body,
        out_type=[
            jax.ShapeDtypeStruct((E, 16), jnp.float32),
            jax.ShapeDtypeStruct((NC, N, 16), jnp.float32),
        ],
        mesh=mesh,
        scratch_types=[
            pltpu.VMEM((CHUNK,), jnp.int32),
            pltpu.VMEM((CHUNK,), jnp.int32),
            pltpu.VMEM((CHUNK, HD), jnp.float32),
            pltpu.VMEM((CHUNK, HD), jnp.float32),
            pltpu.VMEM((CHUNK, HD), jnp.float32),
            pltpu.VMEM((HD,), jnp.float32),
            pltpu.VMEM((CHUNK, 16), jnp.float32),
            pltpu.VMEM((1000, 16), jnp.float32),
            pltpu.VMEM_SHARED((N, 16), jnp.float32),
            pltpu.SemaphoreType.DMA,
            pltpu.SemaphoreType.DMA,
            pltpu.SemaphoreType.DMA,
            pltpu.SemaphoreType.DMA,
        ],
    )
    return f(edge_index, xl, xr, ea, att_flat)


def _sc_pass2_body(ei, xl, s_in, den0, den1, batch, acc_out,
                   srcb, dstb, xlb, sb, d0b, d1b, batchb, accv, idxg,
                   acc_sh, sem1, sem2):
    cid = lax.axis_index("c")
    sid = lax.axis_index("s")
    wid = sid * NC + cid
    base_e = wid * EPW

    pltpu.sync_copy(batch.at[pl.ds(0, N)], batchb)

    def _za(i, _):
        accv[pl.ds(i * 16, 16)] = jnp.zeros((16,), jnp.float32)
        return 0
    lax.fori_loop(0, G * D // 16, _za, 0)

    lane = lax.broadcasted_iota(jnp.int32, (16,), 0)
    for t in range(G // 16):
        idxg[pl.ds(t * 16, 16)] = lane + t * 16

    @pl.when(sid == 0)
    def _():
        def _zs(i, _):
            pltpu.sync_copy(accv.at[pl.ds(i * D, D)], acc_sh.at[i])
            return 0
        lax.fori_loop(0, G, _zs, 0)

    plsc.subcore_barrier()

    def _chunk(k, _):
        gbase = base_e + k * CHUNK
        pltpu.sync_copy(ei.at[0, pl.ds(gbase, CHUNK)], srcb)
        pltpu.sync_copy(ei.at[1, pl.ds(gbase, CHUNK)], dstb)
        c1 = pltpu.async_copy(xl.at[srcb], xlb, sem1)
        pltpu.sync_copy(s_in.at[pl.ds(gbase, CHUNK)], sb)
        c2 = pltpu.async_copy(den0.at[dstb], d0b, sem2)
        c2.wait()
        c3 = pltpu.async_copy(den1.at[dstb], d1b, sem2)
        c3.wait()
        c1.wait()

        def _edge(i, _):
            dv = dstb[i]
            g = batchb[dv]
            gd = g * D
            ws = []
            for h in range(H):
                ws.append(sb[i, h] / (d0b[i, h] + d1b[i, h]))

            def _dd(j, _):
                off = j * 16
                v = ws[0] * xlb[i, pl.ds(off, 16)]
                for h in range(1, H):
                    v = v + ws[h] * xlb[i, pl.ds(h * D + off, 16)]
                accv[pl.ds(gd + off, 16)] = accv[pl.ds(gd + off, 16)] + v
                return 0
            lax.fori_loop(0, D // 16, _dd, 0)
            return 0
        lax.fori_loop(0, CHUNK, _edge, 0)
        return 0

    lax.fori_loop(0, NCHUNK, _chunk, 0)

    acc2 = accv.reshape(G, D)
    pltpu.async_copy(acc2, acc_sh.at[idxg], sem1, add=True).wait()
    plsc.subcore_barrier()

    @pl.when(sid == 0)
    def _():
        def _dump(i, _):
            pltpu.sync_copy(acc_sh.at[i], acc_out.at[cid, i])
            return 0
        lax.fori_loop(0, G, _dump, 0)


def _sc_pass2(edge_index, xl, s_arr, den0, den1, batch):
    mesh = plsc.VectorSubcoreMesh(core_axis_name="c", subcore_axis_name="s",
                                  num_cores=NC, num_subcores=NS)
    f = pl.kernel(
        _sc_pass2_body,
        out_type=jax.ShapeDtypeStruct((NC, G, D), jnp.float32),
        mesh=mesh,
        scratch_types=[
            pltpu.VMEM((CHUNK,), jnp.int32),
            pltpu.VMEM((CHUNK,), jnp.int32),
            pltpu.VMEM((CHUNK, HD), jnp.float32),
            pltpu.VMEM((CHUNK, 16), jnp.float32),
            pltpu.VMEM((CHUNK, 16), jnp.float32),
            pltpu.VMEM((CHUNK, 16), jnp.float32),
            pltpu.VMEM((N,), jnp.int32),
            pltpu.VMEM((G * D,), jnp.float32),
            pltpu.VMEM((G,), jnp.int32),
            pltpu.VMEM_SHARED((G, D), jnp.float32),
            pltpu.SemaphoreType.DMA,
            pltpu.SemaphoreType.DMA,
        ],
    )
    return f(edge_index, xl, s_arr, den0, den1, batch)


def _tail_body(a0_ref, a1_ref, batch_ref, biashd_ref, Wa_ref, ba_ref,
               g1_ref, be1_ref, cb_ref, Wo_ref, bo_ref, g2_ref, be2_ref,
               tok_ref, idx_ref, loss_ref):
    batchv = batch_ref[...]
    gids = lax.broadcasted_iota(jnp.int32, (G, N), 0)
    onehot = (batchv == gids).astype(jnp.float32)
    counts = jnp.sum(onehot, axis=1, keepdims=True)

    accsum = (a0_ref[...] + a1_ref[...]) * (1.0 / H)
    bias_mean = jnp.mean(biashd_ref[...], axis=0, keepdims=True)
    gemb = accsum / jnp.maximum(counts, 1.0) + bias_mean
    gemb = jnp.dot(gemb, Wa_ref[...], preferred_element_type=jnp.float32) + ba_ref[...]
    mu = jnp.mean(gemb, axis=-1, keepdims=True)
    xc = gemb - mu
    var = jnp.mean(xc * xc, axis=-1, keepdims=True)
    gemb = xc / jnp.sqrt(var + 1e-5) * g1_ref[...] + be1_ref[...]

    kio = lax.broadcasted_iota(jnp.int32, (G, K), 1)
    residual = gemb
    quantized = jnp.zeros_like(gemb)
    commits = []
    idx_cols = []
    for q in range(Q):
        cb = cb_ref[q]
        d2 = (jnp.sum(residual * residual, axis=-1, keepdims=True)
              - 2.0 * lax.dot_general(residual, cb, (((1,), (1,)), ((), ())),
                                      preferred_element_type=jnp.float32)
              + jnp.sum(cb * cb, axis=-1)[None, :])
        dmin = jnp.min(d2, axis=-1, keepdims=True)
        cand = jnp.where(d2 <= dmin, kio, K)
        idx = jnp.min(cand, axis=-1)
        oh = (kio == idx[:, None]).astype(jnp.float32)
        qi = jnp.dot(oh, cb, preferred_element_type=jnp.float32)
        commits.append(jnp.mean((qi - residual) ** 2))
        quantized = quantized + qi
        residual = residual - qi
        idx_cols.append(idx)

        t = jnp.dot(qi, Wo_ref[...], preferred_element_type=jnp.float32) + bo_ref[...]
        tmu = jnp.mean(t, axis=-1, keepdims=True)
        tcc = t - tmu
        tvar = jnp.mean(tcc * tcc, axis=-1, keepdims=True)
        t = tcc / jnp.sqrt(tvar + 1e-5) * g2_ref[...] + be2_ref[...]
        tok_ref[:, q, :] = t

    residual_loss = jnp.mean((gemb - quantized) ** 2)
    commit_loss = (commits[0] + commits[1] + commits[2]) / Q
    loss_ref[...] = jnp.broadcast_to(commit_loss + residual_loss, (1, 1))
    idx_ref[...] = jnp.stack(idx_cols, axis=-1)


def _tail(acc0, acc1, batch2d, bias_hd, W_a, b_a, g1, be1, codebooks,
          W_o, b_o, g2, be2):
    return pl.pallas_call(
        _tail_body,
        out_shape=[
            jax.ShapeDtypeStruct((G, Q, FD), jnp.float32),
            jax.ShapeDtypeStruct((G, Q), jnp.int32),
            jax.ShapeDtypeStruct((1, 1), jnp.float32),
        ],
    )(acc0, acc1, batch2d, bias_hd, W_a, b_a.reshape(1, D), g1.reshape(1, D),
      be1.reshape(1, D), codebooks, W_o, b_o.reshape(1, FD), g2.reshape(1, FD),
      be2.reshape(1, FD))


def kernel(x, edge_index, edge_attr, batch, W_l, b_l, W_r, b_r, W_e, att, bias,
           W_a, b_a, g1, be1, codebooks, W_o, b_o, g2, be2):
    xl, xr = _mm2(x, W_l, b_l, W_r, b_r)
    ea = _ea_mm(edge_attr, W_e)
    s_arr, denoms = _sc_pass1(edge_index, xl, xr, ea, att.reshape(HD))
    accs = _sc_pass2(edge_index, xl, s_arr, denoms[0], denoms[1], batch)
    tokens, indices, loss = _tail(
        accs[0], accs[1], batch.reshape(1, N), bias.reshape(H, D),
        W_a, b_a, g1, be1, codebooks, W_o, b_o, g2, be2)
    return tokens, indices, loss.reshape(())


# speedup vs baseline: 4.5578x; 4.5578x over previous
import jax
import jax.numpy as jnp
from jax import lax
from jax.experimental import pallas as pl
from jax.experimental.pallas import tpu as pltpu
from jax.experimental.pallas import tpu_sc as plsc

N = 10000
E = 160000
D = 256
H = 4
ED = 16
FD = 512
Q = 3
K = 512
G = 64
HD = H * D

NC = 2
NS = 16
NW = NC * NS
EPW = E // NW
C1 = 24
NCHUNK1 = EPW // C1
T1 = EPW - C1 * NCHUNK1
CHUNK = 40
NCHUNK = EPW // CHUNK
NZR = N // NS


def _mm2_body(x_ref, wl_ref, bl_ref, wr_ref, br_ref, o1_ref, o2_ref):
    xb = x_ref[...]
    o1_ref[...] = jnp.dot(xb, wl_ref[...], preferred_element_type=jnp.float32) + bl_ref[...]
    o2_ref[...] = jnp.dot(xb, wr_ref[...], preferred_element_type=jnp.float32) + br_ref[...]


def _mm2(x, W_l, b_l, W_r, b_r):
    BN = 2000
    return pl.pallas_call(
        _mm2_body,
        grid=(N // BN,),
        in_specs=[
            pl.BlockSpec((BN, D), lambda i: (i, 0)),
            pl.BlockSpec((D, HD), lambda i: (0, 0)),
            pl.BlockSpec((1, HD), lambda i: (0, 0)),
            pl.BlockSpec((D, HD), lambda i: (0, 0)),
            pl.BlockSpec((1, HD), lambda i: (0, 0)),
        ],
        out_specs=[
            pl.BlockSpec((BN, HD), lambda i: (i, 0)),
            pl.BlockSpec((BN, HD), lambda i: (i, 0)),
        ],
        out_shape=[
            jax.ShapeDtypeStruct((N, HD), jnp.float32),
            jax.ShapeDtypeStruct((N, HD), jnp.float32),
        ],
    )(x, W_l, b_l.reshape(1, HD), W_r, b_r.reshape(1, HD))


def _mm1_body(a_ref, w_ref, o_ref):
    o_ref[...] = jnp.dot(a_ref[...], w_ref[...], preferred_element_type=jnp.float32)


def _ea_mm(edge_attr, W_e):
    BE = 4000
    return pl.pallas_call(
        _mm1_body,
        grid=(E // BE,),
        in_specs=[
            pl.BlockSpec((BE, ED), lambda i: (i, 0)),
            pl.BlockSpec((ED, HD), lambda i: (0, 0)),
        ],
        out_specs=pl.BlockSpec((BE, HD), lambda i: (i, 0)),
        out_shape=jax.ShapeDtypeStruct((E, HD), jnp.float32),
    )(edge_attr, W_e)


def _sc_pass1a_body(ei, xl, xr, ea, att, alpha_out, maxp_out,
                    srcb, dstb, dstp, srcb_t, dstb_t,
                    xlb, xrb, eab, attv, srow, srow_t, flattab,
                    sem1, sem2, sem3):
    cid = lax.axis_index("c")
    sid = lax.axis_index("s")
    wid = sid * NC + cid
    base_e = wid * EPW

    pltpu.sync_copy(att.at[pl.ds(0, HD)], attv)

    NEG = jnp.float32(-3.0e38)

    def _ft(i, _):
        flattab[pl.ds(i * 16, 16)] = jnp.full((16,), NEG, jnp.float32)
        return 0
    lax.fori_loop(0, 4 * N // 16, _ft, 0)

    lane = lax.broadcasted_iota(jnp.int32, (16,), 0)
    hmask = lane < H

    def _emit_chunk(gbase, cn, srcr, dstr, xlr, xrr, ear, srr):
        pltpu.sync_copy(ei.at[0, pl.ds(gbase, cn)], srcr)
        pltpu.sync_copy(ei.at[1, pl.ds(gbase, cn)], dstr)
        pltpu.sync_copy(ei.at[1, pl.ds(gbase, cn)], dstp.at[pl.ds(0, cn)])
        c1 = pltpu.async_copy(xl.at[srcr], xlr, sem1)
        c2 = pltpu.async_copy(xr.at[dstr], xrr, sem2)
        c3 = pltpu.async_copy(ea.at[pl.ds(gbase, cn)], ear, sem3)
        c1.wait()
        c2.wait()
        c3.wait()

        def _edge(i, _):
            svec = jnp.zeros((16,), jnp.float32)
            for h in range(H):
                def _dd(j, acc):
                    off = h * D + j * 16
                    z = xlr[i, pl.ds(off, 16)] + xrr[i, pl.ds(off, 16)] + ear[i, pl.ds(off, 16)]
                    z = jnp.maximum(z, 0.2 * z)
                    return acc + z * attv[pl.ds(off, 16)]
                acc = lax.fori_loop(0, D // 16, _dd, jnp.zeros((16,), jnp.float32))
                svec = svec + jnp.sum(acc) * (lane == h).astype(jnp.float32)
            svec = jnp.where(hmask, svec, 0.0)
            srr[i, :] = svec
            dv = dstp[pl.ds(i, 16)][0]
            idx4 = dv * 4 + lane
            cur = plsc.load_gather(flattab, [idx4], mask=hmask)
            plsc.store_scatter(flattab, [idx4], jnp.maximum(cur, svec), mask=hmask)
            return 0
        lax.fori_loop(0, cn, _edge, 0)

        pltpu.sync_copy(srr, alpha_out.at[pl.ds(gbase, cn)])

    def _chunk(k, _):
        _emit_chunk(base_e + k * C1, C1, srcb, dstb, xlb, xrb, eab, srow)
        return 0

    lax.fori_loop(0, NCHUNK1, _chunk, 0)
    _emit_chunk(base_e + NCHUNK1 * C1, T1, srcb_t, dstb_t,
                xlb.at[pl.ds(0, T1)], xrb.at[pl.ds(0, T1)],
                eab.at[pl.ds(0, T1)], srow_t)

    pltpu.sync_copy(flattab, maxp_out.at[wid])


def _sc_pass1a(edge_index, xl, xr, ea, att_flat):
    mesh = plsc.VectorSubcoreMesh(core_axis_name="c", subcore_axis_name="s",
                                  num_cores=NC, num_subcores=NS)
    f = pl.kernel(
        _sc_pass1a_body,
        out_type=[
            jax.ShapeDtypeStruct((E, 16), jnp.float32),
            jax.ShapeDtypeStruct((NW, 4 * N), jnp.float32),
        ],
        mesh=mesh,
        scratch_types=[
            pltpu.VMEM((C1,), jnp.int32),
            pltpu.VMEM((C1,), jnp.int32),
            pltpu.VMEM((C1 + 16,), jnp.int32),
            pltpu.VMEM((T1,), jnp.int32),
            pltpu.VMEM((T1,), jnp.int32),
            pltpu.VMEM((C1, HD), jnp.float32),
            pltpu.VMEM((C1, HD), jnp.float32),
            pltpu.VMEM((C1, HD), jnp.float32),
            pltpu.VMEM((HD,), jnp.float32),
            pltpu.VMEM((C1, 16), jnp.float32),
            pltpu.VMEM((T1, 16), jnp.float32),
            pltpu.VMEM((4 * N,), jnp.float32),
            pltpu.SemaphoreType.DMA,
            pltpu.SemaphoreType.DMA,
            pltpu.SemaphoreType.DMA,
        ],
        compiler_params=pltpu.CompilerParams(use_tc_tiling_on_sc=False, needs_layout_passes=False),
    )
    return f(edge_index, xl, xr, ea, att_flat)


MS_A = 2512
MS_B = 4 * N - 15 * MS_A

def _sc_pass1b_body(ei, alpha_in, maxp, s_out, denom_out,
                    dstb, dstp, ab, srow, mbuf, macc, amaxv, zrow,
                    amax_sh, denom_sh, sem4):
    cid = lax.axis_index("c")
    sid = lax.axis_index("s")
    wid = sid * NC + cid
    base_e = wid * EPW

    def _merge(off, sz):
        def _mw(w, _):
            pltpu.sync_copy(maxp.at[w, pl.ds(off, sz)], mbuf.at[pl.ds(0, sz)])
            def _mx(j, _):
                macc[pl.ds(j * 16, 16)] = jnp.maximum(
                    macc[pl.ds(j * 16, 16)], mbuf[pl.ds(j * 16, 16)])
                return 0
            lax.fori_loop(0, sz // 16, _mx, 0)
            return 0
        pltpu.sync_copy(maxp.at[0, pl.ds(off, sz)], macc.at[pl.ds(0, sz)])
        lax.fori_loop(1, NW, _mw, 0)
        pltpu.sync_copy(macc.at[pl.ds(0, sz)], amax_sh.at[pl.ds(off, sz)])

    @pl.when(sid < 15)
    def _():
        _merge(sid * MS_A, MS_A)

    @pl.when(sid == 15)
    def _():
        _merge(15 * MS_A, MS_B)

    def _zr(i, _):
        zrow[i, :] = jnp.zeros((16,), jnp.float32)
        return 0
    lax.fori_loop(0, NZR, _zr, 0)
    pltpu.sync_copy(zrow, denom_sh.at[pl.ds(sid * NZR, NZR)])

    plsc.subcore_barrier()

    pltpu.sync_copy(amax_sh, amaxv)

    lane = lax.broadcasted_iota(jnp.int32, (16,), 0)
    hmask = lane < H

    def _chunk(k, _):
        gbase = base_e + k * CHUNK
        pltpu.sync_copy(ei.at[1, pl.ds(gbase, CHUNK)], dstb)
        pltpu.sync_copy(ei.at[1, pl.ds(gbase, CHUNK)], dstp.at[pl.ds(0, CHUNK)])
        pltpu.sync_copy(alpha_in.at[pl.ds(gbase, CHUNK)], ab)

        def _edge(i, _):
            dv = dstp[pl.ds(i, 16)][0]
            idx4 = dv * 4 + lane
            mvec = plsc.load_gather(amaxv, [idx4], mask=hmask)
            av = ab[i, :]
            srow[i, :] = jnp.where(hmask, jnp.exp(av - mvec), 0.0)
            return 0
        lax.fori_loop(0, CHUNK, _edge, 0)

        pltpu.sync_copy(srow, s_out.at[pl.ds(gbase, CHUNK)])
        pltpu.async_copy(srow, denom_sh.at[dstb], sem4, add=True).wait()
        return 0

    lax.fori_loop(0, NCHUNK, _chunk, 0)

    plsc.subcore_barrier()

    pltpu.sync_copy(denom_sh.at[pl.ds(sid * NZR, NZR)],
                    denom_out.at[cid, pl.ds(sid * NZR, NZR)])


def _sc_pass1b(edge_index, alpha_arr, maxp):
    mesh = plsc.VectorSubcoreMesh(core_axis_name="c", subcore_axis_name="s",
                                  num_cores=NC, num_subcores=NS)
    f = pl.kernel(
        _sc_pass1b_body,
        out_type=[
            jax.ShapeDtypeStruct((E, 16), jnp.float32),
            jax.ShapeDtypeStruct((NC, N, 16), jnp.float32),
        ],
        mesh=mesh,
        scratch_types=[
            pltpu.VMEM((CHUNK,), jnp.int32),
            pltpu.VMEM((CHUNK + 16,), jnp.int32),
            pltpu.VMEM((CHUNK, 16), jnp.float32),
            pltpu.VMEM((CHUNK, 16), jnp.float32),
            pltpu.VMEM((MS_A,), jnp.float32),
            pltpu.VMEM((MS_A,), jnp.float32),
            pltpu.VMEM((4 * N,), jnp.float32),
            pltpu.VMEM((NZR, 16), jnp.float32),
            pltpu.VMEM_SHARED((4 * N,), jnp.float32),
            pltpu.VMEM_SHARED((N, 16), jnp.float32),
            pltpu.SemaphoreType.DMA,
        ],
        compiler_params=pltpu.CompilerParams(use_tc_tiling_on_sc=False, needs_layout_passes=False),
    )
    return f(edge_index, alpha_arr, maxp)


def _sc_pass2_body(ei, xl, s_in, den0, den1, batch, acc_out,
                   srcb, dstb, dstp, xlb, sb, d0b, d1b, batchb, accv, idxg,
                   acc_sh, sem1, sem2):
    cid = lax.axis_index("c")
    sid = lax.axis_index("s")
    wid = sid * NC + cid
    base_e = wid * EPW

    pltpu.sync_copy(batch.at[pl.ds(0, N)], batchb.at[pl.ds(0, N)])

    def _za(i, _):
        for j in range(D // 16):
            accv[i, pl.ds(j * 16, 16)] = jnp.zeros((16,), jnp.float32)
        return 0
    lax.fori_loop(0, G, _za, 0)

    lane = lax.broadcasted_iota(jnp.int32, (16,), 0)
    for t in range(G // 16):
        idxg[pl.ds(t * 16, 16)] = lane + t * 16

    @pl.when(sid == 0)
    def _():
        def _zs(i, _):
            pltpu.sync_copy(accv.at[i], acc_sh.at[i])
            return 0
        lax.fori_loop(0, G, _zs, 0)

    plsc.subcore_barrier()

    def _chunk(k, _):
        gbase = base_e + k * CHUNK
        pltpu.sync_copy(ei.at[0, pl.ds(gbase, CHUNK)], srcb)
        pltpu.sync_copy(ei.at[1, pl.ds(gbase, CHUNK)], dstb)
        pltpu.sync_copy(ei.at[1, pl.ds(gbase, CHUNK)], dstp.at[pl.ds(0, CHUNK)])
        c1 = pltpu.async_copy(xl.at[srcb], xlb, sem1)
        pltpu.sync_copy(s_in.at[pl.ds(gbase, CHUNK)], sb)
        c2 = pltpu.async_copy(den0.at[dstb], d0b, sem2)
        c2.wait()
        c3 = pltpu.async_copy(den1.at[dstb], d1b, sem2)
        c3.wait()
        c1.wait()

        def _edge(i, _):
            dv = dstp[pl.ds(i, 16)][0]
            g = batchb[pl.ds(dv, 16)][0]
            wv = sb[i, :] / (d0b[i, :] + d1b[i, :] + 1e-16)
            ws = [wv[h] for h in range(H)]

            def _dd(j, _):
                off = j * 16
                v = ws[0] * xlb[i, pl.ds(off, 16)]
                for h in range(1, H):
                    v = v + ws[h] * xlb[i, pl.ds(h * D + off, 16)]
                accv[g, pl.ds(off, 16)] = accv[g, pl.ds(off, 16)] + v
                return 0
            lax.fori_loop(0, D // 16, _dd, 0)
            return 0
        lax.fori_loop(0, CHUNK, _edge, 0)
        return 0

    lax.fori_loop(0, NCHUNK, _chunk, 0)

    pltpu.async_copy(accv, acc_sh.at[idxg], sem1, add=True).wait()
    plsc.subcore_barrier()

    @pl.when(sid == 0)
    def _():
        def _dump(i, _):
            pltpu.sync_copy(acc_sh.at[i], acc_out.at[cid, i])
            return 0
        lax.fori_loop(0, G, _dump, 0)


def _sc_pass2(edge_index, xl, s_arr, den0, den1, batch):
    mesh = plsc.VectorSubcoreMesh(core_axis_name="c", subcore_axis_name="s",
                                  num_cores=NC, num_subcores=NS)
    f = pl.kernel(
        _sc_pass2_body,
        out_type=jax.ShapeDtypeStruct((NC, G, D), jnp.float32),
        mesh=mesh,
        scratch_types=[
            pltpu.VMEM((CHUNK,), jnp.int32),
            pltpu.VMEM((CHUNK,), jnp.int32),
            pltpu.VMEM((CHUNK + 16,), jnp.int32),
            pltpu.VMEM((CHUNK, HD), jnp.float32),
            pltpu.VMEM((CHUNK, 16), jnp.float32),
            pltpu.VMEM((CHUNK, 16), jnp.float32),
            pltpu.VMEM((CHUNK, 16), jnp.float32),
            pltpu.VMEM((N + 16,), jnp.int32),
            pltpu.VMEM((G, D), jnp.float32),
            pltpu.VMEM((G,), jnp.int32),
            pltpu.VMEM_SHARED((G, D), jnp.float32),
            pltpu.SemaphoreType.DMA,
            pltpu.SemaphoreType.DMA,
        ],
        compiler_params=pltpu.CompilerParams(use_tc_tiling_on_sc=False, needs_layout_passes=False),
    )
    return f(edge_index, xl, s_arr, den0, den1, batch)


def _tail_body(a0_ref, a1_ref, batch_ref, biashd_ref, Wa_ref, ba_ref,
               g1_ref, be1_ref, cb_ref, Wo_ref, bo_ref, g2_ref, be2_ref,
               tok_ref, idx_ref, loss_ref):
    batchv = batch_ref[...]
    gids = lax.broadcasted_iota(jnp.int32, (G, N), 0)
    onehot = (batchv == gids).astype(jnp.float32)
    counts = jnp.sum(onehot, axis=1, keepdims=True)

    accsum = (a0_ref[...] + a1_ref[...]) * (1.0 / H)
    bias_mean = jnp.mean(biashd_ref[...], axis=0, keepdims=True)
    gemb = accsum / jnp.maximum(counts, 1.0) + bias_mean
    gemb = jnp.dot(gemb, Wa_ref[...], preferred_element_type=jnp.float32) + ba_ref[...]
    mu = jnp.mean(gemb, axis=-1, keepdims=True)
    xc = gemb - mu
    var = jnp.mean(xc * xc, axis=-1, keepdims=True)
    gemb = xc / jnp.sqrt(var + 1e-5) * g1_ref[...] + be1_ref[...]

    kio = lax.broadcasted_iota(jnp.int32, (G, K), 1)
    residual = gemb
    quantized = jnp.zeros_like(gemb)
    commits = []
    idx_cols = []
    for q in range(Q):
        cb = cb_ref[q]
        d2 = (jnp.sum(residual * residual, axis=-1, keepdims=True)
              - 2.0 * lax.dot_general(residual, cb, (((1,), (1,)), ((), ())),
                                      preferred_element_type=jnp.float32)
              + jnp.sum(cb * cb, axis=-1)[None, :])
        dmin = jnp.min(d2, axis=-1, keepdims=True)
        cand = jnp.where(d2 <= dmin, kio, K)
        idx = jnp.min(cand, axis=-1)
        oh = (kio == idx[:, None]).astype(jnp.float32)
        qi = jnp.dot(oh, cb, preferred_element_type=jnp.float32,
                     precision=lax.Precision.HIGHEST)
        commits.append(jnp.mean((qi - residual) ** 2))
        quantized = quantized + qi
        residual = residual - qi
        idx_cols.append(idx)

        t = jnp.dot(qi, Wo_ref[...], preferred_element_type=jnp.float32) + bo_ref[...]
        tmu = jnp.mean(t, axis=-1, keepdims=True)
        tcc = t - tmu
        tvar = jnp.mean(tcc * tcc, axis=-1, keepdims=True)
        t = tcc / jnp.sqrt(tvar + 1e-5) * g2_ref[...] + be2_ref[...]
        tok_ref[:, q, :] = t

    residual_loss = jnp.mean((gemb - quantized) ** 2)
    commit_loss = (commits[0] + commits[1] + commits[2]) / Q
    loss_ref[...] = jnp.broadcast_to(commit_loss + residual_loss, (1, 1))
    idx_ref[...] = jnp.stack(idx_cols, axis=-1)


def _tail(acc0, acc1, batch2d, bias_hd, W_a, b_a, g1, be1, codebooks,
          W_o, b_o, g2, be2):
    return pl.pallas_call(
        _tail_body,
        out_shape=[
            jax.ShapeDtypeStruct((G, Q, FD), jnp.float32),
            jax.ShapeDtypeStruct((G, Q), jnp.int32),
            jax.ShapeDtypeStruct((1, 1), jnp.float32),
        ],
    )(acc0, acc1, batch2d, bias_hd, W_a, b_a.reshape(1, D), g1.reshape(1, D),
      be1.reshape(1, D), codebooks, W_o, b_o.reshape(1, FD),
      g2.reshape(1, FD), be2.reshape(1, FD))


def kernel(x, edge_index, edge_attr, batch, W_l, b_l, W_r, b_r, W_e, att, bias,
           W_a, b_a, g1, be1, codebooks, W_o, b_o, g2, be2):
    xl, xr = _mm2(x, W_l, b_l, W_r, b_r)
    ea = _ea_mm(edge_attr, W_e)
    alpha_arr, maxp = _sc_pass1a(edge_index, xl, xr, ea, att.reshape(HD))
    s_arr, denoms = _sc_pass1b(edge_index, alpha_arr, maxp)
    accs = _sc_pass2(edge_index, xl, s_arr, denoms[0], denoms[1], batch)
    tokens, indices, loss = _tail(
        accs[0], accs[1], batch.reshape(1, N), bias.reshape(H, D),
        W_a, b_a, g1, be1, codebooks, W_o, b_o, g2, be2)
    return tokens, indices, loss.reshape(())


# speedup vs baseline: 4.7789x; 1.0485x over previous
import jax
import jax.numpy as jnp
from jax import lax
from jax.experimental import pallas as pl
from jax.experimental.pallas import tpu as pltpu
from jax.experimental.pallas import tpu_sc as plsc

N = 10000
E = 160000
D = 256
H = 4
ED = 16
FD = 512
Q = 3
K = 512
G = 64
HD = H * D

NC = 2
NS = 16
NW = NC * NS
EPW = E // NW
C1 = 24
NCHUNK1 = EPW // C1
T1 = EPW - C1 * NCHUNK1
CHUNK = 40
NCHUNK = EPW // CHUNK
NZR = N // NS


def _mm2_body(x_ref, wl_ref, bl_ref, wr_ref, br_ref, o1_ref, o2_ref):
    xb = x_ref[...]
    o1_ref[...] = jnp.dot(xb, wl_ref[...], preferred_element_type=jnp.float32) + bl_ref[...]
    o2_ref[...] = jnp.dot(xb, wr_ref[...], preferred_element_type=jnp.float32) + br_ref[...]


def _mm2(x, W_l, b_l, W_r, b_r):
    BN = 2000
    return pl.pallas_call(
        _mm2_body,
        grid=(N // BN,),
        in_specs=[
            pl.BlockSpec((BN, D), lambda i: (i, 0)),
            pl.BlockSpec((D, HD), lambda i: (0, 0)),
            pl.BlockSpec((1, HD), lambda i: (0, 0)),
            pl.BlockSpec((D, HD), lambda i: (0, 0)),
            pl.BlockSpec((1, HD), lambda i: (0, 0)),
        ],
        out_specs=[
            pl.BlockSpec((BN, HD), lambda i: (i, 0)),
            pl.BlockSpec((BN, HD), lambda i: (i, 0)),
        ],
        out_shape=[
            jax.ShapeDtypeStruct((N, HD), jnp.float32),
            jax.ShapeDtypeStruct((N, HD), jnp.float32),
        ],
    )(x, W_l, b_l.reshape(1, HD), W_r, b_r.reshape(1, HD))


def _mm1_body(a_ref, w_ref, o_ref):
    o_ref[...] = jnp.dot(a_ref[...], w_ref[...], preferred_element_type=jnp.float32)


def _ea_mm(edge_attr, W_e):
    BE = 4000
    return pl.pallas_call(
        _mm1_body,
        grid=(E // BE,),
        in_specs=[
            pl.BlockSpec((BE, ED), lambda i: (i, 0)),
            pl.BlockSpec((ED, HD), lambda i: (0, 0)),
        ],
        out_specs=pl.BlockSpec((BE, HD), lambda i: (i, 0)),
        out_shape=jax.ShapeDtypeStruct((E, HD), jnp.float32),
    )(edge_attr, W_e)


def _sc_pass1a_body(ei, xl, xr, ea, att, alpha_out, maxp_out,
                    srcb, dstb, dstp, srcb_t, dstb_t,
                    xlb, xrb, eab, attv, srow, srow_t, flattab,
                    sem1, sem2, sem3):
    cid = lax.axis_index("c")
    sid = lax.axis_index("s")
    wid = sid * NC + cid
    base_e = wid * EPW

    pltpu.sync_copy(att.at[pl.ds(0, HD)], attv)

    NEG = jnp.float32(-3.0e38)

    def _ft(i, _):
        flattab[pl.ds(i * 16, 16)] = jnp.full((16,), NEG, jnp.float32)
        return 0
    lax.fori_loop(0, 4 * N // 16, _ft, 0)

    lane = lax.broadcasted_iota(jnp.int32, (16,), 0)
    hmask = lane < H

    def _emit_chunk(gbase, cn, srcr, dstr, xlr, xrr, ear, srr):
        pltpu.sync_copy(ei.at[0, pl.ds(gbase, cn)], srcr)
        pltpu.sync_copy(ei.at[1, pl.ds(gbase, cn)], dstr)
        pltpu.sync_copy(ei.at[1, pl.ds(gbase, cn)], dstp.at[pl.ds(0, cn)])
        c1 = pltpu.async_copy(xl.at[srcr], xlr, sem1)
        c2 = pltpu.async_copy(xr.at[dstr], xrr, sem2)
        c3 = pltpu.async_copy(ea.at[pl.ds(gbase, cn)], ear, sem3)
        c1.wait()
        c2.wait()
        c3.wait()

        def _edge(i, _):
            svec = jnp.zeros((16,), jnp.float32)
            for h in range(H):
                acc = jnp.zeros((16,), jnp.float32)
                for j in range(D // 16):
                    off = h * D + j * 16
                    z = xlr[i, pl.ds(off, 16)] + xrr[i, pl.ds(off, 16)] + ear[i, pl.ds(off, 16)]
                    z = jnp.maximum(z, 0.2 * z)
                    acc = acc + z * attv[pl.ds(off, 16)]
                svec = svec + jnp.sum(acc) * (lane == h).astype(jnp.float32)
            svec = jnp.where(hmask, svec, 0.0)
            srr[i, :] = svec
            dv = dstp[pl.ds(i, 16)][0]
            idx4 = dv * 4 + lane
            cur = plsc.load_gather(flattab, [idx4], mask=hmask)
            plsc.store_scatter(flattab, [idx4], jnp.maximum(cur, svec), mask=hmask)
            return 0
        lax.fori_loop(0, cn, _edge, 0)

        pltpu.sync_copy(srr, alpha_out.at[pl.ds(gbase, cn)])

    def _chunk(k, _):
        _emit_chunk(base_e + k * C1, C1, srcb, dstb, xlb, xrb, eab, srow)
        return 0

    lax.fori_loop(0, NCHUNK1, _chunk, 0)
    _emit_chunk(base_e + NCHUNK1 * C1, T1, srcb_t, dstb_t,
                xlb.at[pl.ds(0, T1)], xrb.at[pl.ds(0, T1)],
                eab.at[pl.ds(0, T1)], srow_t)

    pltpu.sync_copy(flattab, maxp_out.at[wid])


def _sc_pass1a(edge_index, xl, xr, ea, att_flat):
    mesh = plsc.VectorSubcoreMesh(core_axis_name="c", subcore_axis_name="s",
                                  num_cores=NC, num_subcores=NS)
    f = pl.kernel(
        _sc_pass1a_body,
        out_type=[
            jax.ShapeDtypeStruct((E, 16), jnp.float32),
            jax.ShapeDtypeStruct((NW, 4 * N), jnp.float32),
        ],
        mesh=mesh,
        scratch_types=[
            pltpu.VMEM((C1,), jnp.int32),
            pltpu.VMEM((C1,), jnp.int32),
            pltpu.VMEM((C1 + 16,), jnp.int32),
            pltpu.VMEM((T1,), jnp.int32),
            pltpu.VMEM((T1,), jnp.int32),
            pltpu.VMEM((C1, HD), jnp.float32),
            pltpu.VMEM((C1, HD), jnp.float32),
            pltpu.VMEM((C1, HD), jnp.float32),
            pltpu.VMEM((HD,), jnp.float32),
            pltpu.VMEM((C1, 16), jnp.float32),
            pltpu.VMEM((T1, 16), jnp.float32),
            pltpu.VMEM((4 * N,), jnp.float32),
            pltpu.SemaphoreType.DMA,
            pltpu.SemaphoreType.DMA,
            pltpu.SemaphoreType.DMA,
        ],
        compiler_params=pltpu.CompilerParams(use_tc_tiling_on_sc=False, needs_layout_passes=False),
    )
    return f(edge_index, xl, xr, ea, att_flat)


MS_A = 2512
MS_B = 4 * N - 15 * MS_A

def _sc_pass1b_body(ei, alpha_in, maxp, s_out, denom_out,
                    dstb, dstp, ab, srow, mbuf, macc, amaxv, zrow,
                    amax_sh, denom_sh, sem4):
    cid = lax.axis_index("c")
    sid = lax.axis_index("s")
    wid = sid * NC + cid
    base_e = wid * EPW

    def _merge(off, sz):
        def _mw(w, _):
            pltpu.sync_copy(maxp.at[w, pl.ds(off, sz)], mbuf.at[pl.ds(0, sz)])
            def _mx(j, _):
                macc[pl.ds(j * 16, 16)] = jnp.maximum(
                    macc[pl.ds(j * 16, 16)], mbuf[pl.ds(j * 16, 16)])
                return 0
            lax.fori_loop(0, sz // 16, _mx, 0)
            return 0
        pltpu.sync_copy(maxp.at[0, pl.ds(off, sz)], macc.at[pl.ds(0, sz)])
        lax.fori_loop(1, NW, _mw, 0)
        pltpu.sync_copy(macc.at[pl.ds(0, sz)], amax_sh.at[pl.ds(off, sz)])

    @pl.when(sid < 15)
    def _():
        _merge(sid * MS_A, MS_A)

    @pl.when(sid == 15)
    def _():
        _merge(15 * MS_A, MS_B)

    def _zr(i, _):
        zrow[i, :] = jnp.zeros((16,), jnp.float32)
        return 0
    lax.fori_loop(0, NZR, _zr, 0)
    pltpu.sync_copy(zrow, denom_sh.at[pl.ds(sid * NZR, NZR)])

    plsc.subcore_barrier()

    pltpu.sync_copy(amax_sh, amaxv)

    lane = lax.broadcasted_iota(jnp.int32, (16,), 0)
    hmask = lane < H

    def _chunk(k, _):
        gbase = base_e + k * CHUNK
        pltpu.sync_copy(ei.at[1, pl.ds(gbase, CHUNK)], dstb)
        pltpu.sync_copy(ei.at[1, pl.ds(gbase, CHUNK)], dstp.at[pl.ds(0, CHUNK)])
        pltpu.sync_copy(alpha_in.at[pl.ds(gbase, CHUNK)], ab)

        def _edge(i, _):
            dv = dstp[pl.ds(i, 16)][0]
            idx4 = dv * 4 + lane
            mvec = plsc.load_gather(amaxv, [idx4], mask=hmask)
            av = ab[i, :]
            srow[i, :] = jnp.where(hmask, jnp.exp(av - mvec), 0.0)
            return 0
        lax.fori_loop(0, CHUNK, _edge, 0)

        pltpu.sync_copy(srow, s_out.at[pl.ds(gbase, CHUNK)])
        pltpu.async_copy(srow, denom_sh.at[dstb], sem4, add=True).wait()
        return 0

    lax.fori_loop(0, NCHUNK, _chunk, 0)

    plsc.subcore_barrier()

    pltpu.sync_copy(denom_sh.at[pl.ds(sid * NZR, NZR)],
                    denom_out.at[cid, pl.ds(sid * NZR, NZR)])


def _sc_pass1b(edge_index, alpha_arr, maxp):
    mesh = plsc.VectorSubcoreMesh(core_axis_name="c", subcore_axis_name="s",
                                  num_cores=NC, num_subcores=NS)
    f = pl.kernel(
        _sc_pass1b_body,
        out_type=[
            jax.ShapeDtypeStruct((E, 16), jnp.float32),
            jax.ShapeDtypeStruct((NC, N, 16), jnp.float32),
        ],
        mesh=mesh,
        scratch_types=[
            pltpu.VMEM((CHUNK,), jnp.int32),
            pltpu.VMEM((CHUNK + 16,), jnp.int32),
            pltpu.VMEM((CHUNK, 16), jnp.float32),
            pltpu.VMEM((CHUNK, 16), jnp.float32),
            pltpu.VMEM((MS_A,), jnp.float32),
            pltpu.VMEM((MS_A,), jnp.float32),
            pltpu.VMEM((4 * N,), jnp.float32),
            pltpu.VMEM((NZR, 16), jnp.float32),
            pltpu.VMEM_SHARED((4 * N,), jnp.float32),
            pltpu.VMEM_SHARED((N, 16), jnp.float32),
            pltpu.SemaphoreType.DMA,
        ],
        compiler_params=pltpu.CompilerParams(use_tc_tiling_on_sc=False, needs_layout_passes=False),
    )
    return f(edge_index, alpha_arr, maxp)


def _sc_pass2_body(ei, xl, s_in, den0, den1, batch, acc_out,
                   srcb, dstb, dstp, xlb, sb, d0b, d1b, batchb, accv, idxg,
                   acc_sh, sem1, sem2):
    cid = lax.axis_index("c")
    sid = lax.axis_index("s")
    wid = sid * NC + cid
    base_e = wid * EPW

    pltpu.sync_copy(batch.at[pl.ds(0, N)], batchb.at[pl.ds(0, N)])

    def _za(i, _):
        for j in range(D // 16):
            accv[i, pl.ds(j * 16, 16)] = jnp.zeros((16,), jnp.float32)
        return 0
    lax.fori_loop(0, G, _za, 0)

    lane = lax.broadcasted_iota(jnp.int32, (16,), 0)
    for t in range(G // 16):
        idxg[pl.ds(t * 16, 16)] = lane + t * 16

    @pl.when(sid == 0)
    def _():
        def _zs(i, _):
            pltpu.sync_copy(accv.at[i], acc_sh.at[i])
            return 0
        lax.fori_loop(0, G, _zs, 0)

    plsc.subcore_barrier()

    def _chunk(k, _):
        gbase = base_e + k * CHUNK
        pltpu.sync_copy(ei.at[0, pl.ds(gbase, CHUNK)], srcb)
        pltpu.sync_copy(ei.at[1, pl.ds(gbase, CHUNK)], dstb)
        pltpu.sync_copy(ei.at[1, pl.ds(gbase, CHUNK)], dstp.at[pl.ds(0, CHUNK)])
        c1 = pltpu.async_copy(xl.at[srcb], xlb, sem1)
        pltpu.sync_copy(s_in.at[pl.ds(gbase, CHUNK)], sb)
        c2 = pltpu.async_copy(den0.at[dstb], d0b, sem2)
        c2.wait()
        c3 = pltpu.async_copy(den1.at[dstb], d1b, sem2)
        c3.wait()
        c1.wait()

        def _edge(i, _):
            dv = dstp[pl.ds(i, 16)][0]
            g = batchb[pl.ds(dv, 16)][0]
            wv = sb[i, :] / (d0b[i, :] + d1b[i, :] + 1e-16)
            ws = [wv[h] for h in range(H)]

            for j in range(D // 16):
                off = j * 16
                v = ws[0] * xlb[i, pl.ds(off, 16)]
                for h in range(1, H):
                    v = v + ws[h] * xlb[i, pl.ds(h * D + off, 16)]
                accv[g, pl.ds(off, 16)] = accv[g, pl.ds(off, 16)] + v
            return 0
        lax.fori_loop(0, CHUNK, _edge, 0)
        return 0

    lax.fori_loop(0, NCHUNK, _chunk, 0)

    pltpu.async_copy(accv, acc_sh.at[idxg], sem1, add=True).wait()
    plsc.subcore_barrier()

    @pl.when(sid == 0)
    def _():
        def _dump(i, _):
            pltpu.sync_copy(acc_sh.at[i], acc_out.at[cid, i])
            return 0
        lax.fori_loop(0, G, _dump, 0)


def _sc_pass2(edge_index, xl, s_arr, den0, den1, batch):
    mesh = plsc.VectorSubcoreMesh(core_axis_name="c", subcore_axis_name="s",
                                  num_cores=NC, num_subcores=NS)
    f = pl.kernel(
        _sc_pass2_body,
        out_type=jax.ShapeDtypeStruct((NC, G, D), jnp.float32),
        mesh=mesh,
        scratch_types=[
            pltpu.VMEM((CHUNK,), jnp.int32),
            pltpu.VMEM((CHUNK,), jnp.int32),
            pltpu.VMEM((CHUNK + 16,), jnp.int32),
            pltpu.VMEM((CHUNK, HD), jnp.float32),
            pltpu.VMEM((CHUNK, 16), jnp.float32),
            pltpu.VMEM((CHUNK, 16), jnp.float32),
            pltpu.VMEM((CHUNK, 16), jnp.float32),
            pltpu.VMEM((N + 16,), jnp.int32),
            pltpu.VMEM((G, D), jnp.float32),
            pltpu.VMEM((G,), jnp.int32),
            pltpu.VMEM_SHARED((G, D), jnp.float32),
            pltpu.SemaphoreType.DMA,
            pltpu.SemaphoreType.DMA,
        ],
        compiler_params=pltpu.CompilerParams(use_tc_tiling_on_sc=False, needs_layout_passes=False),
    )
    return f(edge_index, xl, s_arr, den0, den1, batch)


def _tail_body(a0_ref, a1_ref, batch_ref, biashd_ref, Wa_ref, ba_ref,
               g1_ref, be1_ref, cb_ref, Wo_ref, bo_ref, g2_ref, be2_ref,
               tok_ref, idx_ref, loss_ref):
    batchv = batch_ref[...]
    gids = lax.broadcasted_iota(jnp.int32, (G, N), 0)
    onehot = (batchv == gids).astype(jnp.float32)
    counts = jnp.sum(onehot, axis=1, keepdims=True)

    accsum = (a0_ref[...] + a1_ref[...]) * (1.0 / H)
    bias_mean = jnp.mean(biashd_ref[...], axis=0, keepdims=True)
    gemb = accsum / jnp.maximum(counts, 1.0) + bias_mean
    gemb = jnp.dot(gemb, Wa_ref[...], preferred_element_type=jnp.float32) + ba_ref[...]
    mu = jnp.mean(gemb, axis=-1, keepdims=True)
    xc = gemb - mu
    var = jnp.mean(xc * xc, axis=-1, keepdims=True)
    gemb = xc / jnp.sqrt(var + 1e-5) * g1_ref[...] + be1_ref[...]

    kio = lax.broadcasted_iota(jnp.int32, (G, K), 1)
    residual = gemb
    quantized = jnp.zeros_like(gemb)
    commits = []
    idx_cols = []
    for q in range(Q):
        cb = cb_ref[q]
        d2 = (jnp.sum(residual * residual, axis=-1, keepdims=True)
              - 2.0 * lax.dot_general(residual, cb, (((1,), (1,)), ((), ())),
                                      preferred_element_type=jnp.float32)
              + jnp.sum(cb * cb, axis=-1)[None, :])
        dmin = jnp.min(d2, axis=-1, keepdims=True)
        cand = jnp.where(d2 <= dmin, kio, K)
        idx = jnp.min(cand, axis=-1)
        oh = (kio == idx[:, None]).astype(jnp.float32)
        qi = jnp.dot(oh, cb, preferred_element_type=jnp.float32,
                     precision=lax.Precision.HIGHEST)
        commits.append(jnp.mean((qi - residual) ** 2))
        quantized = quantized + qi
        residual = residual - qi
        idx_cols.append(idx)

        t = jnp.dot(qi, Wo_ref[...], preferred_element_type=jnp.float32) + bo_ref[...]
        tmu = jnp.mean(t, axis=-1, keepdims=True)
        tcc = t - tmu
        tvar = jnp.mean(tcc * tcc, axis=-1, keepdims=True)
        t = tcc / jnp.sqrt(tvar + 1e-5) * g2_ref[...] + be2_ref[...]
        tok_ref[:, q, :] = t

    residual_loss = jnp.mean((gemb - quantized) ** 2)
    commit_loss = (commits[0] + commits[1] + commits[2]) / Q
    loss_ref[...] = jnp.broadcast_to(commit_loss + residual_loss, (1, 1))
    idx_ref[...] = jnp.stack(idx_cols, axis=-1)


def _tail(acc0, acc1, batch2d, bias_hd, W_a, b_a, g1, be1, codebooks,
          W_o, b_o, g2, be2):
    return pl.pallas_call(
        _tail_body,
        out_shape=[
            jax.ShapeDtypeStruct((G, Q, FD), jnp.float32),
            jax.ShapeDtypeStruct((G, Q), jnp.int32),
            jax.ShapeDtypeStruct((1, 1), jnp.float32),
        ],
    )(acc0, acc1, batch2d, bias_hd, W_a, b_a.reshape(1, D), g1.reshape(1, D),
      be1.reshape(1, D), codebooks, W_o, b_o.reshape(1, FD),
      g2.reshape(1, FD), be2.reshape(1, FD))


def kernel(x, edge_index, edge_attr, batch, W_l, b_l, W_r, b_r, W_e, att, bias,
           W_a, b_a, g1, be1, codebooks, W_o, b_o, g2, be2):
    xl, xr = _mm2(x, W_l, b_l, W_r, b_r)
    ea = _ea_mm(edge_attr, W_e)
    alpha_arr, maxp = _sc_pass1a(edge_index, xl, xr, ea, att.reshape(HD))
    s_arr, denoms = _sc_pass1b(edge_index, alpha_arr, maxp)
    accs = _sc_pass2(edge_index, xl, s_arr, denoms[0], denoms[1], batch)
    tokens, indices, loss = _tail(
        accs[0], accs[1], batch.reshape(1, N), bias.reshape(H, D),
        W_a, b_a, g1, be1, codebooks, W_o, b_o, g2, be2)
    return tokens, indices, loss.reshape(())
